# Initial kernel scaffold; baseline (speedup 1.0000x reference)
import jax
import jax.numpy as jnp
from jax.experimental import pallas as pl


def kernel(weight, ids):
    raise NotImplementedError("write your pallas kernel here")


# speedup vs baseline: 3.8999x; 3.8999x over previous
import math

import jax
import jax.numpy as jnp
from jax.experimental import pallas as pl
from jax.experimental.pallas import tpu as pltpu

_TN = 64


def _vmem_gather_kernel(ids_ref, w_ref, o_ref):
    base = pl.program_id(0) * _TN
    for t in range(_TN):
        o_ref[t, 0] = w_ref[ids_ref[base + t], 0]


def kernel(weight, ids):
    ids_shape = ids.shape
    V, D = weight.shape
    N = math.prod(ids_shape)
    flat_ids = ids.reshape(N).astype(jnp.int32)

    npad = (-N) % _TN
    if npad:
        flat_ids = jnp.pad(flat_ids, (0, npad))
    Np = N + npad

    w3 = weight.reshape(V, 1, D)

    out = pl.pallas_call(
        _vmem_gather_kernel,
        out_shape=jax.ShapeDtypeStruct((Np, 1, D), weight.dtype),
        grid_spec=pltpu.PrefetchScalarGridSpec(
            num_scalar_prefetch=1,
            grid=(Np // _TN,),
            in_specs=[
                pl.BlockSpec((V, 1, D), lambda i, ids: (0, 0, 0)),
            ],
            out_specs=pl.BlockSpec((_TN, 1, D), lambda i, ids: (i, 0, 0)),
        ),
        compiler_params=pltpu.CompilerParams(
            dimension_semantics=("parallel",),
        ),
    )(flat_ids, w3)

    out = out.reshape(Np, D)
    if npad:
        out = out[:N]
    return out.reshape(*ids_shape, D)


# speedup vs baseline: 6.8555x; 1.7579x over previous
import math

import jax
import jax.numpy as jnp
from jax.experimental import pallas as pl
from jax.experimental.pallas import tpu as pltpu

_TN = 256


def _vmem_gather_kernel(ids_ref, w_ref, o_ref):
    base = pl.program_id(0) * _TN
    for t in range(_TN):
        o_ref[t, 0] = w_ref[ids_ref[base + t], 0]


def kernel(weight, ids):
    ids_shape = ids.shape
    V, D = weight.shape
    N = math.prod(ids_shape)
    flat_ids = ids.reshape(N).astype(jnp.int32)

    npad = (-N) % _TN
    if npad:
        flat_ids = jnp.pad(flat_ids, (0, npad))
    Np = N + npad

    w3 = weight.reshape(V, 1, D)

    out = pl.pallas_call(
        _vmem_gather_kernel,
        out_shape=jax.ShapeDtypeStruct((Np, 1, D), weight.dtype),
        grid_spec=pltpu.PrefetchScalarGridSpec(
            num_scalar_prefetch=1,
            grid=(Np // _TN,),
            in_specs=[
                pl.BlockSpec((V, 1, D), lambda i, ids: (0, 0, 0)),
            ],
            out_specs=pl.BlockSpec((_TN, 1, D), lambda i, ids: (i, 0, 0)),
        ),
        compiler_params=pltpu.CompilerParams(
            dimension_semantics=("parallel",),
        ),
    )(flat_ids, w3)

    out = out.reshape(Np, D)
    if npad:
        out = out[:N]
    return out.reshape(*ids_shape, D)
